# baseline (speedup 1.0000x reference)
import functools

import jax
import jax.numpy as jnp
from jax import lax
from jax.experimental import pallas as pl
from jax.experimental.pallas import tpu as pltpu
from jax.experimental.pallas import tpu_sc as plsc

N = 10000
E = 320000
N_PAD = 10240
NC = 2
NS = 16
CHUNK = 80
ROWS_PER_TILE = N_PAD // NS

_f32 = jnp.float32
_i32 = jnp.int32


def _sc_mesh():
    return plsc.VectorSubcoreMesh(
        core_axis_name="c", subcore_axis_name="s", num_cores=NC,
        num_subcores=NS)


def _zero_rows(buf, nrows):
    def zrow(i, _):
        for j in range(128 // 16):
            buf[i, pl.ds(j * 16, 16)] = jnp.zeros((16,), _f32)
        return 0
    lax.fori_loop(0, nrows, zrow, 0)


PER_TILE = E // (NC * NS)
NCHUNKS = PER_TILE // CHUNK


def _copy_idx(dst, src, i):
    for j in range(CHUNK // 16):
        dst[pl.ds(j * 16, 16)] = src[pl.ds(i * CHUNK + j * 16, 16)]


@functools.partial(
    pl.kernel,
    out_type=jax.ShapeDtypeStruct((NC, N_PAD, 128), _f32),
    mesh=_sc_mesh(),
    scratch_types=[
        pltpu.VMEM((CHUNK, 128), _f32),
        pltpu.VMEM((PER_TILE,), _i32),
        pltpu.VMEM((CHUNK,), _i32),
        pltpu.VMEM_SHARED((N_PAD, 128), _f32),
        pltpu.SemaphoreType.DMA,
    ],
)
def _sc_deg(col_hbm, deg_out, ones_v, call_v, idx_v, deg_sh, sem):
    cid = lax.axis_index("c")
    sid = lax.axis_index("s")
    base = cid * (E // NC) + sid * PER_TILE
    ld = pltpu.async_copy(col_hbm.at[pl.ds(base, PER_TILE)], call_v, sem)

    _zero_rows(ones_v, CHUNK)
    for k in range(ROWS_PER_TILE // CHUNK):
        pltpu.sync_copy(
            ones_v, deg_sh.at[pl.ds(sid * ROWS_PER_TILE + k * CHUNK, CHUNK)])

    def orow(i, _):
        for j in range(128 // 16):
            ones_v[i, pl.ds(j * 16, 16)] = jnp.full((16,), 1.0, _f32)
        return 0
    lax.fori_loop(0, CHUNK, orow, 0)
    ld.wait()
    plsc.subcore_barrier()

    def step(i, _):
        _copy_idx(idx_v, call_v, i)
        pltpu.sync_copy(ones_v, deg_sh.at[idx_v], add=True)
        return 0
    lax.fori_loop(0, NCHUNKS, step, 0)
    plsc.subcore_barrier()

    for k in range(ROWS_PER_TILE // CHUNK):
        r0 = sid * ROWS_PER_TILE + k * CHUNK
        pltpu.sync_copy(deg_sh.at[pl.ds(r0, CHUNK)],
                        deg_out.at[cid, pl.ds(r0, CHUNK)])


ACHUNK = 40
ANCH = PER_TILE // ACHUNK
NBUF = 8
GAHEAD = NBUF - 2


@functools.partial(
    pl.kernel,
    out_type=jax.ShapeDtypeStruct((NC, N_PAD, 128), _f32),
    mesh=_sc_mesh(),
    scratch_types=(
        [pltpu.VMEM((ACHUNK, 128), _f32) for _ in range(NBUF)] +
        [pltpu.VMEM((ACHUNK,), _i32) for _ in range(NBUF)] +
        [pltpu.VMEM((ACHUNK,), _i32) for _ in range(NBUF)] +
        [pltpu.VMEM_SHARED((N_PAD, 128), _f32)] +
        [pltpu.SemaphoreType.DMA for _ in range(2 * NBUF)]
    ),
)
def _sc_agg(row_hbm, col_hbm, hs_hbm, out_hbm, *refs):
    gbufs = refs[0:NBUF]
    rbufs = refs[NBUF:2 * NBUF]
    cbufs = refs[2 * NBUF:3 * NBUF]
    acc_sh = refs[3 * NBUF]
    gsems = refs[3 * NBUF + 1:3 * NBUF + 1 + NBUF]
    isems = refs[3 * NBUF + 1 + NBUF:]

    cid = lax.axis_index("c")
    sid = lax.axis_index("s")
    base = cid * (E // NC) + sid * PER_TILE

    def load_idx(i, b):
        e0 = base + i * ACHUNK
        pltpu.async_copy(row_hbm.at[pl.ds(e0, ACHUNK)], rbufs[b], isems[b])
        pltpu.async_copy(col_hbm.at[pl.ds(e0, ACHUNK)], cbufs[b], isems[b])

    def wait_idx(b):
        pltpu.make_async_copy(row_hbm.at[pl.ds(0, ACHUNK)], rbufs[b],
                              isems[b]).wait()
        pltpu.make_async_copy(col_hbm.at[pl.ds(0, ACHUNK)], cbufs[b],
                              isems[b]).wait()

    def issue_gather(b):
        pltpu.async_copy(hs_hbm.at[rbufs[b]], gbufs[b], gsems[b])

    def drain_scatter(b):
        pltpu.make_async_copy(hs_hbm.at[rbufs[b]], gbufs[b], gsems[b]).wait()
        pltpu.sync_copy(gbufs[b], acc_sh.at[cbufs[b]], add=True)

    for j in range(NBUF - 1):
        load_idx(j, j)

    _zero_rows(gbufs[NBUF - 1], ACHUNK)
    for k in range(ROWS_PER_TILE // ACHUNK):
        pltpu.sync_copy(
            gbufs[NBUF - 1],
            acc_sh.at[pl.ds(sid * ROWS_PER_TILE + k * ACHUNK, ACHUNK)])
    plsc.subcore_barrier()

    for j in range(GAHEAD):
        wait_idx(j)
        issue_gather(j)

    def body(g, _):
        for k in range(NBUF):
            i = NBUF * g + k

            @pl.when(i + NBUF - 1 < ANCH)
            def _():
                load_idx(i + NBUF - 1, (k + NBUF - 1) % NBUF)

            @pl.when(i + GAHEAD < ANCH)
            def _():
                wait_idx((k + GAHEAD) % NBUF)
                issue_gather((k + GAHEAD) % NBUF)

            @pl.when(i < ANCH)
            def _():
                drain_scatter(k)
        return 0
    lax.fori_loop(0, (ANCH + NBUF - 1) // NBUF, body, 0)
    plsc.subcore_barrier()

    for k in range(ROWS_PER_TILE // ACHUNK):
        r0 = sid * ROWS_PER_TILE + k * ACHUNK
        pltpu.sync_copy(acc_sh.at[pl.ds(r0, ACHUNK)],
                        out_hbm.at[cid, pl.ds(r0, ACHUNK)])


BN = 400


def _hs1_body(x_ref, deg_ref, w_ref, o_ref):
    dinv = lax.rsqrt(deg_ref[0] + deg_ref[1] + 1.0)
    h = jnp.dot(x_ref[...], w_ref[...], preferred_element_type=_f32)
    o_ref[...] = h * dinv


def _tc_hs1(x, deg_parts, W1):
    return pl.pallas_call(
        _hs1_body,
        grid=(N // BN,),
        in_specs=[pl.BlockSpec((BN, 128), lambda i: (i, 0)),
                  pl.BlockSpec((2, BN, 128), lambda i: (0, i, 0)),
                  pl.BlockSpec((128, 128), lambda i: (0, 0))],
        out_specs=pl.BlockSpec((BN, 128), lambda i: (i, 0)),
        out_shape=jax.ShapeDtypeStruct((N_PAD, 128), _f32),
    )(x, deg_parts, W1)


def _mid_body(agg_ref, hs_ref, deg_ref, w_ref, b_ref, o_ref):
    dinv = lax.rsqrt(deg_ref[0] + deg_ref[1] + 1.0)
    s = agg_ref[0] + agg_ref[1] + hs_ref[...]
    h1 = jnp.maximum(dinv * s + b_ref[...], 0.0)
    hs2 = jnp.dot(h1, w_ref[...], preferred_element_type=_f32) * dinv[:, :64]
    o_ref[...] = jnp.concatenate(
        [hs2, jnp.zeros((BN, 64), _f32)], axis=1)


def _tc_mid(agg1, hs1, deg_parts, W2, b1):
    return pl.pallas_call(
        _mid_body,
        grid=(N // BN,),
        in_specs=[pl.BlockSpec((2, BN, 128), lambda i: (0, i, 0)),
                  pl.BlockSpec((BN, 128), lambda i: (i, 0)),
                  pl.BlockSpec((2, BN, 128), lambda i: (0, i, 0)),
                  pl.BlockSpec((128, 64), lambda i: (0, 0)),
                  pl.BlockSpec((1, 128), lambda i: (0, 0))],
        out_specs=pl.BlockSpec((BN, 128), lambda i: (i, 0)),
        out_shape=jax.ShapeDtypeStruct((N_PAD, 128), _f32),
    )(agg1, hs1, deg_parts, W2, b1)


def _out_body(agg_ref, hs_ref, deg_ref, b_ref, o_ref):
    dinv = lax.rsqrt(deg_ref[0] + deg_ref[1] + 1.0)
    s = agg_ref[0] + agg_ref[1] + hs_ref[...]
    o_ref[...] = dinv[:, :64] * s[:, :64] + b_ref[...]


def _tc_out(agg2, hs2, deg_parts, b2):
    return pl.pallas_call(
        _out_body,
        grid=(N // BN,),
        in_specs=[pl.BlockSpec((2, BN, 128), lambda i: (0, i, 0)),
                  pl.BlockSpec((BN, 128), lambda i: (i, 0)),
                  pl.BlockSpec((2, BN, 128), lambda i: (0, i, 0)),
                  pl.BlockSpec((1, 64), lambda i: (0, 0))],
        out_specs=pl.BlockSpec((BN, 64), lambda i: (i, 0)),
        out_shape=jax.ShapeDtypeStruct((N, 64), _f32),
    )(agg2, hs2, deg_parts, b2)


def kernel(x, edge_index, W1, b1, W2, b2):
    row = edge_index[0]
    col = edge_index[1]

    deg_parts = _sc_deg(col)
    hs1 = _tc_hs1(x, deg_parts, W1)
    agg1 = _sc_agg(row, col, hs1)
    hs2 = _tc_mid(agg1, hs1, deg_parts, W2,
                  b1.reshape(1, 128))
    agg2 = _sc_agg(row, col, hs2)
    return _tc_out(agg2, hs2, deg_parts, b2.reshape(1, 64))


# speedup vs baseline: 1.1435x; 1.1435x over previous
import functools

import jax
import jax.numpy as jnp
from jax import lax
from jax.experimental import pallas as pl
from jax.experimental.pallas import tpu as pltpu
from jax.experimental.pallas import tpu_sc as plsc

N = 10000
E = 320000
N_PAD = 10240
NC = 2
NS = 16
CHUNK = 80
ROWS_PER_TILE = N_PAD // NS

_f32 = jnp.float32
_i32 = jnp.int32


def _sc_mesh():
    return plsc.VectorSubcoreMesh(
        core_axis_name="c", subcore_axis_name="s", num_cores=NC,
        num_subcores=NS)


def _zero_rows(buf, nrows):
    def zrow(i, _):
        for j in range(128 // 16):
            buf[i, pl.ds(j * 16, 16)] = jnp.zeros((16,), _f32)
        return 0
    lax.fori_loop(0, nrows, zrow, 0)


PER_TILE = E // (NC * NS)
NCHUNKS = PER_TILE // CHUNK


def _copy_idx(dst, src, i):
    for j in range(CHUNK // 16):
        dst[pl.ds(j * 16, 16)] = src[pl.ds(i * CHUNK + j * 16, 16)]


@functools.partial(
    pl.kernel,
    out_type=jax.ShapeDtypeStruct((NC, N_PAD, 128), _f32),
    mesh=_sc_mesh(),
    scratch_types=[
        pltpu.VMEM((CHUNK, 128), _f32),
        pltpu.VMEM((PER_TILE,), _i32),
        pltpu.VMEM((CHUNK,), _i32),
        pltpu.VMEM_SHARED((N_PAD, 128), _f32),
        pltpu.SemaphoreType.DMA,
    ],
)
def _sc_deg(col_hbm, deg_out, ones_v, call_v, idx_v, deg_sh, sem):
    cid = lax.axis_index("c")
    sid = lax.axis_index("s")
    base = cid * (E // NC) + sid * PER_TILE
    ld = pltpu.async_copy(col_hbm.at[pl.ds(base, PER_TILE)], call_v, sem)

    _zero_rows(ones_v, CHUNK)
    for k in range(ROWS_PER_TILE // CHUNK):
        pltpu.sync_copy(
            ones_v, deg_sh.at[pl.ds(sid * ROWS_PER_TILE + k * CHUNK, CHUNK)])

    def orow(i, _):
        for j in range(128 // 16):
            ones_v[i, pl.ds(j * 16, 16)] = jnp.full((16,), 1.0, _f32)
        return 0
    lax.fori_loop(0, CHUNK, orow, 0)
    ld.wait()
    plsc.subcore_barrier()

    def step(i, _):
        _copy_idx(idx_v, call_v, i)
        pltpu.sync_copy(ones_v, deg_sh.at[idx_v], add=True)
        return 0
    lax.fori_loop(0, NCHUNKS, step, 0)
    plsc.subcore_barrier()

    for k in range(ROWS_PER_TILE // CHUNK):
        r0 = sid * ROWS_PER_TILE + k * CHUNK
        pltpu.sync_copy(deg_sh.at[pl.ds(r0, CHUNK)],
                        deg_out.at[cid, pl.ds(r0, CHUNK)])


NBUF = 4
IBUF = 8
GAHEAD = 3


@functools.partial(
    pl.kernel,
    out_type=jax.ShapeDtypeStruct((NC, N_PAD, 128), _f32),
    mesh=_sc_mesh(),
    scratch_types=(
        [pltpu.VMEM((CHUNK, 128), _f32) for _ in range(NBUF)] +
        [pltpu.VMEM((CHUNK,), _i32) for _ in range(IBUF)] +
        [pltpu.VMEM((CHUNK,), _i32) for _ in range(IBUF)] +
        [pltpu.VMEM_SHARED((N_PAD, 128), _f32)] +
        [pltpu.SemaphoreType.DMA for _ in range(NBUF + IBUF)]
    ),
)
def _sc_agg(row_hbm, col_hbm, hs_hbm, out_hbm, *refs):
    gbufs = refs[0:NBUF]
    rbufs = refs[NBUF:NBUF + IBUF]
    cbufs = refs[NBUF + IBUF:NBUF + 2 * IBUF]
    acc_sh = refs[NBUF + 2 * IBUF]
    gsems = refs[NBUF + 2 * IBUF + 1:NBUF + 2 * IBUF + 1 + NBUF]
    isems = refs[NBUF + 2 * IBUF + 1 + NBUF:]

    cid = lax.axis_index("c")
    sid = lax.axis_index("s")
    base = cid * (E // NC) + sid * PER_TILE

    def load_idx(i, b):
        e0 = base + i * CHUNK
        pltpu.async_copy(row_hbm.at[pl.ds(e0, CHUNK)], rbufs[b], isems[b])
        pltpu.async_copy(col_hbm.at[pl.ds(e0, CHUNK)], cbufs[b], isems[b])

    def wait_idx(b):
        pltpu.make_async_copy(row_hbm.at[pl.ds(0, CHUNK)], rbufs[b],
                              isems[b]).wait()
        pltpu.make_async_copy(col_hbm.at[pl.ds(0, CHUNK)], cbufs[b],
                              isems[b]).wait()

    def issue_gather(bi, bg):
        pltpu.async_copy(hs_hbm.at[rbufs[bi]], gbufs[bg], gsems[bg])

    def drain_scatter(bi, bg):
        pltpu.make_async_copy(hs_hbm.at[rbufs[bi]], gbufs[bg],
                              gsems[bg]).wait()
        pltpu.sync_copy(gbufs[bg], acc_sh.at[cbufs[bi]], add=True)

    for j in range(IBUF - 1):
        load_idx(j, j)

    _zero_rows(gbufs[NBUF - 1], CHUNK)
    for k in range(ROWS_PER_TILE // CHUNK):
        pltpu.sync_copy(
            gbufs[NBUF - 1],
            acc_sh.at[pl.ds(sid * ROWS_PER_TILE + k * CHUNK, CHUNK)])
    plsc.subcore_barrier()

    for j in range(GAHEAD):
        wait_idx(j)
        issue_gather(j, j)

    def body(g, _):
        for k in range(IBUF):
            i = IBUF * g + k

            @pl.when(i + IBUF - 1 < NCHUNKS)
            def _():
                load_idx(i + IBUF - 1, (k + IBUF - 1) % IBUF)

            @pl.when(i + GAHEAD < NCHUNKS)
            def _():
                wait_idx((k + GAHEAD) % IBUF)
                issue_gather((k + GAHEAD) % IBUF, (k + GAHEAD) % NBUF)

            @pl.when(i < NCHUNKS)
            def _():
                drain_scatter(k, k % NBUF)
        return 0
    lax.fori_loop(0, (NCHUNKS + IBUF - 1) // IBUF, body, 0)
    plsc.subcore_barrier()

    for k in range(ROWS_PER_TILE // CHUNK):
        r0 = sid * ROWS_PER_TILE + k * CHUNK
        pltpu.sync_copy(acc_sh.at[pl.ds(r0, CHUNK)],
                        out_hbm.at[cid, pl.ds(r0, CHUNK)])


BN = 400


def _hs1_body(x_ref, deg_ref, w_ref, o_ref):
    dinv = lax.rsqrt(deg_ref[0] + deg_ref[1] + 1.0)
    h = jnp.dot(x_ref[...], w_ref[...], preferred_element_type=_f32)
    o_ref[...] = h * dinv


def _tc_hs1(x, deg_parts, W1):
    return pl.pallas_call(
        _hs1_body,
        grid=(N // BN,),
        in_specs=[pl.BlockSpec((BN, 128), lambda i: (i, 0)),
                  pl.BlockSpec((2, BN, 128), lambda i: (0, i, 0)),
                  pl.BlockSpec((128, 128), lambda i: (0, 0))],
        out_specs=pl.BlockSpec((BN, 128), lambda i: (i, 0)),
        out_shape=jax.ShapeDtypeStruct((N_PAD, 128), _f32),
    )(x, deg_parts, W1)


def _mid_body(agg_ref, hs_ref, deg_ref, w_ref, b_ref, o_ref):
    dinv = lax.rsqrt(deg_ref[0] + deg_ref[1] + 1.0)
    s = agg_ref[0] + agg_ref[1] + hs_ref[...]
    h1 = jnp.maximum(dinv * s + b_ref[...], 0.0)
    hs2 = jnp.dot(h1, w_ref[...], preferred_element_type=_f32) * dinv[:, :64]
    o_ref[...] = jnp.concatenate(
        [hs2, jnp.zeros((BN, 64), _f32)], axis=1)


def _tc_mid(agg1, hs1, deg_parts, W2, b1):
    return pl.pallas_call(
        _mid_body,
        grid=(N // BN,),
        in_specs=[pl.BlockSpec((2, BN, 128), lambda i: (0, i, 0)),
                  pl.BlockSpec((BN, 128), lambda i: (i, 0)),
                  pl.BlockSpec((2, BN, 128), lambda i: (0, i, 0)),
                  pl.BlockSpec((128, 64), lambda i: (0, 0)),
                  pl.BlockSpec((1, 128), lambda i: (0, 0))],
        out_specs=pl.BlockSpec((BN, 128), lambda i: (i, 0)),
        out_shape=jax.ShapeDtypeStruct((N_PAD, 128), _f32),
    )(agg1, hs1, deg_parts, W2, b1)


def _out_body(agg_ref, hs_ref, deg_ref, b_ref, o_ref):
    dinv = lax.rsqrt(deg_ref[0] + deg_ref[1] + 1.0)
    s = agg_ref[0] + agg_ref[1] + hs_ref[...]
    o_ref[...] = dinv[:, :64] * s[:, :64] + b_ref[...]


def _tc_out(agg2, hs2, deg_parts, b2):
    return pl.pallas_call(
        _out_body,
        grid=(N // BN,),
        in_specs=[pl.BlockSpec((2, BN, 128), lambda i: (0, i, 0)),
                  pl.BlockSpec((BN, 128), lambda i: (i, 0)),
                  pl.BlockSpec((2, BN, 128), lambda i: (0, i, 0)),
                  pl.BlockSpec((1, 64), lambda i: (0, 0))],
        out_specs=pl.BlockSpec((BN, 64), lambda i: (i, 0)),
        out_shape=jax.ShapeDtypeStruct((N, 64), _f32),
    )(agg2, hs2, deg_parts, b2)


def kernel(x, edge_index, W1, b1, W2, b2):
    row = edge_index[0]
    col = edge_index[1]

    deg_parts = _sc_deg(col)
    hs1 = _tc_hs1(x, deg_parts, W1)
    agg1 = _sc_agg(row, col, hs1)
    hs2 = _tc_mid(agg1, hs1, deg_parts, W2,
                  b1.reshape(1, 128))
    agg2 = _sc_agg(row, col, hs2)
    return _tc_out(agg2, hs2, deg_parts, b2.reshape(1, 64))


# speedup vs baseline: 1.1530x; 1.0082x over previous
import functools

import jax
import jax.numpy as jnp
from jax import lax
from jax.experimental import pallas as pl
from jax.experimental.pallas import tpu as pltpu
from jax.experimental.pallas import tpu_sc as plsc

N = 10000
E = 320000
N_PAD = 10240
NC = 2
NS = 16
CHUNK = 80
ROWS_PER_TILE = N_PAD // NS

_f32 = jnp.float32
_i32 = jnp.int32


def _sc_mesh():
    return plsc.VectorSubcoreMesh(
        core_axis_name="c", subcore_axis_name="s", num_cores=NC,
        num_subcores=NS)


def _zero_rows(buf, nrows):
    def zrow(i, _):
        for j in range(128 // 16):
            buf[i, pl.ds(j * 16, 16)] = jnp.zeros((16,), _f32)
        return 0
    lax.fori_loop(0, nrows, zrow, 0)


PER_TILE = E // (NC * NS)
NCHUNKS = PER_TILE // CHUNK


def _copy_idx(dst, src, i):
    for j in range(CHUNK // 16):
        dst[pl.ds(j * 16, 16)] = src[pl.ds(i * CHUNK + j * 16, 16)]


@functools.partial(
    pl.kernel,
    out_type=jax.ShapeDtypeStruct((NC, N_PAD, 128), _f32),
    mesh=_sc_mesh(),
    scratch_types=[
        pltpu.VMEM((CHUNK, 128), _f32),
        pltpu.VMEM((PER_TILE,), _i32),
        pltpu.VMEM((CHUNK,), _i32),
        pltpu.VMEM((CHUNK,), _i32),
        pltpu.VMEM((CHUNK,), _i32),
        pltpu.VMEM((CHUNK,), _i32),
        pltpu.VMEM_SHARED((N_PAD, 128), _f32),
        pltpu.SemaphoreType.DMA,
        pltpu.SemaphoreType.DMA,
        pltpu.SemaphoreType.DMA,
        pltpu.SemaphoreType.DMA,
        pltpu.SemaphoreType.DMA,
    ],
)
def _sc_deg(col_hbm, deg_out, ones_v, call_v, ib0, ib1, ib2, ib3, deg_sh,
            sem, sd0, sd1, sd2, sd3):
    cid = lax.axis_index("c")
    sid = lax.axis_index("s")
    base = cid * (E // NC) + sid * PER_TILE
    ld = pltpu.async_copy(col_hbm.at[pl.ds(base, PER_TILE)], call_v, sem)

    _zero_rows(ones_v, CHUNK)
    for k in range(ROWS_PER_TILE // CHUNK):
        pltpu.sync_copy(
            ones_v, deg_sh.at[pl.ds(sid * ROWS_PER_TILE + k * CHUNK, CHUNK)])

    def orow(i, _):
        for j in range(128 // 16):
            ones_v[i, pl.ds(j * 16, 16)] = jnp.full((16,), 1.0, _f32)
        return 0
    lax.fori_loop(0, CHUNK, orow, 0)
    ld.wait()
    plsc.subcore_barrier()

    ibufs = (ib0, ib1, ib2, ib3)
    dsems = (sd0, sd1, sd2, sd3)

    def drain(k):
        pltpu.make_async_copy(ones_v, deg_sh.at[ibufs[k]], dsems[k]).wait()

    def step(g, _):
        for k in range(4):
            i = 4 * g + k

            @pl.when(jnp.logical_and(i >= 4, i < NCHUNKS))
            def _():
                drain(k)

            @pl.when(i < NCHUNKS)
            def _():
                _copy_idx(ibufs[k], call_v, i)
                pltpu.async_copy(ones_v, deg_sh.at[ibufs[k]], dsems[k],
                                 add=True)
        return 0
    lax.fori_loop(0, (NCHUNKS + 3) // 4, step, 0)
    for k in range(4):
        drain(k)
    plsc.subcore_barrier()

    for k in range(ROWS_PER_TILE // CHUNK):
        r0 = sid * ROWS_PER_TILE + k * CHUNK
        pltpu.sync_copy(deg_sh.at[pl.ds(r0, CHUNK)],
                        deg_out.at[cid, pl.ds(r0, CHUNK)])


NBUF = 4
IBUF = 8
GAHEAD = 3


@functools.partial(
    pl.kernel,
    out_type=jax.ShapeDtypeStruct((NC, N_PAD, 128), _f32),
    mesh=_sc_mesh(),
    scratch_types=(
        [pltpu.VMEM((CHUNK, 128), _f32) for _ in range(NBUF)] +
        [pltpu.VMEM((CHUNK,), _i32) for _ in range(IBUF)] +
        [pltpu.VMEM((CHUNK,), _i32) for _ in range(IBUF)] +
        [pltpu.VMEM_SHARED((N_PAD, 128), _f32)] +
        [pltpu.SemaphoreType.DMA for _ in range(NBUF + IBUF)]
    ),
)
def _sc_agg(row_hbm, col_hbm, hs_hbm, out_hbm, *refs):
    gbufs = refs[0:NBUF]
    rbufs = refs[NBUF:NBUF + IBUF]
    cbufs = refs[NBUF + IBUF:NBUF + 2 * IBUF]
    acc_sh = refs[NBUF + 2 * IBUF]
    gsems = refs[NBUF + 2 * IBUF + 1:NBUF + 2 * IBUF + 1 + NBUF]
    isems = refs[NBUF + 2 * IBUF + 1 + NBUF:]

    cid = lax.axis_index("c")
    sid = lax.axis_index("s")
    base = cid * (E // NC) + sid * PER_TILE

    def load_idx(i, b):
        e0 = base + i * CHUNK
        pltpu.async_copy(row_hbm.at[pl.ds(e0, CHUNK)], rbufs[b], isems[b])
        pltpu.async_copy(col_hbm.at[pl.ds(e0, CHUNK)], cbufs[b], isems[b])

    def wait_idx(b):
        pltpu.make_async_copy(row_hbm.at[pl.ds(0, CHUNK)], rbufs[b],
                              isems[b]).wait()
        pltpu.make_async_copy(col_hbm.at[pl.ds(0, CHUNK)], cbufs[b],
                              isems[b]).wait()

    def issue_gather(bi, bg):
        pltpu.async_copy(hs_hbm.at[rbufs[bi]], gbufs[bg], gsems[bg])

    def drain_scatter(bi, bg):
        pltpu.make_async_copy(hs_hbm.at[rbufs[bi]], gbufs[bg],
                              gsems[bg]).wait()
        pltpu.sync_copy(gbufs[bg], acc_sh.at[cbufs[bi]], add=True)

    for j in range(IBUF - 1):
        load_idx(j, j)

    _zero_rows(gbufs[NBUF - 1], CHUNK)
    for k in range(ROWS_PER_TILE // CHUNK):
        pltpu.sync_copy(
            gbufs[NBUF - 1],
            acc_sh.at[pl.ds(sid * ROWS_PER_TILE + k * CHUNK, CHUNK)])
    plsc.subcore_barrier()

    for j in range(GAHEAD):
        wait_idx(j)
        issue_gather(j, j)

    def body(g, _):
        for k in range(IBUF):
            i = IBUF * g + k

            @pl.when(i + IBUF - 1 < NCHUNKS)
            def _():
                load_idx(i + IBUF - 1, (k + IBUF - 1) % IBUF)

            @pl.when(i + GAHEAD < NCHUNKS)
            def _():
                wait_idx((k + GAHEAD) % IBUF)
                issue_gather((k + GAHEAD) % IBUF, (k + GAHEAD) % NBUF)

            @pl.when(i < NCHUNKS)
            def _():
                drain_scatter(k, k % NBUF)
        return 0
    lax.fori_loop(0, (NCHUNKS + IBUF - 1) // IBUF, body, 0)
    plsc.subcore_barrier()

    for k in range(ROWS_PER_TILE // CHUNK):
        r0 = sid * ROWS_PER_TILE + k * CHUNK
        pltpu.sync_copy(acc_sh.at[pl.ds(r0, CHUNK)],
                        out_hbm.at[cid, pl.ds(r0, CHUNK)])


BN = 400


def _hs1_body(x_ref, deg_ref, w_ref, o_ref):
    dinv = lax.rsqrt(deg_ref[0] + deg_ref[1] + 1.0)
    h = jnp.dot(x_ref[...], w_ref[...], preferred_element_type=_f32)
    o_ref[...] = h * dinv


def _tc_hs1(x, deg_parts, W1):
    return pl.pallas_call(
        _hs1_body,
        grid=(N // BN,),
        in_specs=[pl.BlockSpec((BN, 128), lambda i: (i, 0)),
                  pl.BlockSpec((2, BN, 128), lambda i: (0, i, 0)),
                  pl.BlockSpec((128, 128), lambda i: (0, 0))],
        out_specs=pl.BlockSpec((BN, 128), lambda i: (i, 0)),
        out_shape=jax.ShapeDtypeStruct((N_PAD, 128), _f32),
    )(x, deg_parts, W1)


def _mid_body(agg_ref, hs_ref, deg_ref, w_ref, b_ref, o_ref):
    dinv = lax.rsqrt(deg_ref[0] + deg_ref[1] + 1.0)
    s = agg_ref[0] + agg_ref[1] + hs_ref[...]
    h1 = jnp.maximum(dinv * s + b_ref[...], 0.0)
    hs2 = jnp.dot(h1, w_ref[...], preferred_element_type=_f32) * dinv[:, :64]
    o_ref[...] = jnp.concatenate(
        [hs2, jnp.zeros((BN, 64), _f32)], axis=1)


def _tc_mid(agg1, hs1, deg_parts, W2, b1):
    return pl.pallas_call(
        _mid_body,
        grid=(N // BN,),
        in_specs=[pl.BlockSpec((2, BN, 128), lambda i: (0, i, 0)),
                  pl.BlockSpec((BN, 128), lambda i: (i, 0)),
                  pl.BlockSpec((2, BN, 128), lambda i: (0, i, 0)),
                  pl.BlockSpec((128, 64), lambda i: (0, 0)),
                  pl.BlockSpec((1, 128), lambda i: (0, 0))],
        out_specs=pl.BlockSpec((BN, 128), lambda i: (i, 0)),
        out_shape=jax.ShapeDtypeStruct((N_PAD, 128), _f32),
    )(agg1, hs1, deg_parts, W2, b1)


def _out_body(agg_ref, hs_ref, deg_ref, b_ref, o_ref):
    dinv = lax.rsqrt(deg_ref[0] + deg_ref[1] + 1.0)
    s = agg_ref[0] + agg_ref[1] + hs_ref[...]
    o_ref[...] = dinv[:, :64] * s[:, :64] + b_ref[...]


def _tc_out(agg2, hs2, deg_parts, b2):
    return pl.pallas_call(
        _out_body,
        grid=(N // BN,),
        in_specs=[pl.BlockSpec((2, BN, 128), lambda i: (0, i, 0)),
                  pl.BlockSpec((BN, 128), lambda i: (i, 0)),
                  pl.BlockSpec((2, BN, 128), lambda i: (0, i, 0)),
                  pl.BlockSpec((1, 64), lambda i: (0, 0))],
        out_specs=pl.BlockSpec((BN, 64), lambda i: (i, 0)),
        out_shape=jax.ShapeDtypeStruct((N, 64), _f32),
    )(agg2, hs2, deg_parts, b2)


def kernel(x, edge_index, W1, b1, W2, b2):
    row = edge_index[0]
    col = edge_index[1]

    deg_parts = _sc_deg(col)
    hs1 = _tc_hs1(x, deg_parts, W1)
    agg1 = _sc_agg(row, col, hs1)
    hs2 = _tc_mid(agg1, hs1, deg_parts, W2,
                  b1.reshape(1, 128))
    agg2 = _sc_agg(row, col, hs2)
    return _tc_out(agg2, hs2, deg_parts, b2.reshape(1, 64))
